# Initial kernel scaffold; baseline (speedup 1.0000x reference)
import jax
import jax.numpy as jnp
from jax.experimental import pallas as pl


def kernel(x, table, W, b):
    raise NotImplementedError("write your pallas kernel here")


# speedup vs baseline: 1.0691x; 1.0691x over previous
import functools

import jax
import jax.numpy as jnp
from jax import lax
from jax.experimental import pallas as pl
from jax.experimental.pallas import tpu as pltpu
from jax.experimental.pallas import tpu_sc as plsc

VOCAB = 1000000
EMBED_DIM = 64
BATCH = 4096
HIST = 200

NC = 2
NS = 16
NW = NC * NS
RPW = BATCH // NW
LANES = 16
NG = RPW // LANES

TW_BLOCK = 8192


def _tw_body(t_ref, w_ref, o_ref):
    o_ref[...] = lax.dot_general(
        w_ref[...], t_ref[...], (((1,), (1,)), ((), ())),
        preferred_element_type=jnp.float32)


def _compute_tw(table, W):
    grid = (VOCAB + TW_BLOCK - 1) // TW_BLOCK
    tw2d = pl.pallas_call(
        _tw_body,
        grid=(grid,),
        in_specs=[
            pl.BlockSpec((TW_BLOCK, EMBED_DIM), lambda i: (i, 0)),
            pl.BlockSpec((1, EMBED_DIM), lambda i: (0, 0)),
        ],
        out_specs=pl.BlockSpec((1, TW_BLOCK), lambda i: (0, i)),
        out_shape=jax.ShapeDtypeStruct((1, VOCAB), jnp.float32),
    )(table, W)
    return tw2d.reshape(VOCAB)


IPW = RPW * HIST


def _sc_pool(tw_hbm, xt_hbm, b_hbm, out_hbm, idx_v, vals_v, out_v, b_v, sem):
    wid = lax.axis_index("s") * NC + lax.axis_index("c")
    pltpu.sync_copy(xt_hbm.at[pl.ds(wid * IPW, IPW)], idx_v)
    pltpu.sync_copy(b_hbm, b_v)
    pltpu.async_copy(tw_hbm.at[idx_v], vals_v, sem).wait()
    bvec = b_v[...]
    for g in range(NG):

        def body(j, carry, g=g):
            acc, cnt = carry
            off = j * RPW + g * LANES
            v = vals_v[pl.ds(off, LANES)]
            iv = idx_v[pl.ds(off, LANES)]
            nz = iv != 0
            acc = acc + jnp.where(nz, v, 0.0)
            cnt = cnt + jnp.where(nz, 1.0, 0.0)
            return acc, cnt

        zero = jnp.zeros((LANES,), jnp.float32)
        acc, cnt = lax.fori_loop(0, HIST, body, (zero, zero))
        out_v[pl.ds(g * LANES, LANES)] = acc / cnt + bvec
    pltpu.sync_copy(out_v, out_hbm.at[pl.ds(wid * RPW, RPW)])


def kernel(x, table, W, b):
    tw = _compute_tw(table, W)
    xt = x.reshape(NW, RPW, HIST).transpose(0, 2, 1).reshape(NW * IPW)
    b16 = jnp.broadcast_to(b, (LANES,))

    pooled = functools.partial(
        pl.kernel,
        mesh=plsc.VectorSubcoreMesh(core_axis_name="c", subcore_axis_name="s"),
        out_type=jax.ShapeDtypeStruct((BATCH,), jnp.float32),
        scratch_types=[
            pltpu.VMEM((IPW,), jnp.int32),
            pltpu.VMEM((IPW,), jnp.float32),
            pltpu.VMEM((RPW,), jnp.float32),
            pltpu.VMEM((LANES,), jnp.float32),
            pltpu.SemaphoreType.DMA,
        ],
    )(_sc_pool)(tw, xt, b16)
    return pooled.reshape(BATCH, 1)
